# Initial kernel scaffold; baseline (speedup 1.0000x reference)
import jax
import jax.numpy as jnp
from jax.experimental import pallas as pl


def kernel(x, edge_index, edge_weight, W1, b1, W2, b2):
    raise NotImplementedError("write your pallas kernel here")


# speedup vs baseline: 12.4680x; 12.4680x over previous
import functools

import jax
import jax.numpy as jnp
from jax import lax
from jax.experimental import pallas as pl
from jax.experimental.pallas import tpu as pltpu
from jax.experimental.pallas import tpu_sc as plsc

N_NODES = 10000
D_FEAT = 128
D_HID = 16
N_CLASSES = 7

NC = 2
NS = 16
L = 16
NW = NC * NS
CHUNK = 128
NBUF = 4
N_PAD = 10240
ROWS_PER_SUB = N_PAD // NS


def _make_sc_spmm(K):
  mesh = plsc.VectorSubcoreMesh(core_axis_name="c", subcore_axis_name="s")

  @functools.partial(
      pl.kernel,
      out_type=jax.ShapeDtypeStruct((NC, N_PAD, D_HID), jnp.float32),
      mesh=mesh,
      scratch_types=[
          pltpu.VMEM((K, CHUNK), jnp.int32),
          pltpu.VMEM((K, CHUNK), jnp.int32),
          pltpu.VMEM((K * CHUNK,), jnp.float32),
          pltpu.VMEM((NBUF, CHUNK, D_HID), jnp.float32),
          pltpu.VMEM((ROWS_PER_SUB, D_HID), jnp.float32),
          pltpu.VMEM_SHARED((N_PAD, D_HID), jnp.float32),
          pltpu.SemaphoreType.DMA((NBUF,)),
      ],
      compiler_params=pltpu.CompilerParams(needs_layout_passes=False,
                                           use_tc_tiling_on_sc=False),
  )
  def sc_spmm(h_hbm, src_hbm, dst_hbm, ew_hbm, zeros_hbm, out_hbm,
              src_v, dst_v, ew_v, rows_v, zbuf, acc, gsem):
    c = lax.axis_index("c")
    s = lax.axis_index("s")
    wid = c * NS + s

    pltpu.sync_copy(src_hbm.at[wid], src_v)
    pltpu.sync_copy(dst_hbm.at[wid], dst_v)
    pltpu.sync_copy(ew_hbm.at[wid], ew_v)

    row0 = s * ROWS_PER_SUB
    pltpu.sync_copy(zeros_hbm.at[pl.ds(row0, ROWS_PER_SUB)], zbuf)
    pltpu.sync_copy(zbuf, acc.at[pl.ds(row0, ROWS_PER_SUB)])
    plsc.subcore_barrier()

    for b in range(NBUF):
      pltpu.async_copy(h_hbm.at[src_v.at[b]], rows_v.at[b], gsem.at[b])

    def group(g, carry):
      for b in range(NBUF):
        i = g * NBUF + b
        pltpu.make_async_copy(
            h_hbm.at[src_v.at[i]], rows_v.at[b], gsem.at[b]).wait()
        ibase = i * CHUNK

        def rowbody(j, carry2, b=b, ibase=ibase):
          w = plsc.load_gather(ew_v, [jnp.full((L,), ibase + j, jnp.int32)])
          rows_v[b, j, :] = rows_v[b, j, :] * w
          return carry2

        lax.fori_loop(0, CHUNK, rowbody, 0, unroll=8)
        pltpu.sync_copy(rows_v.at[b], acc.at[dst_v.at[i]], add=True)

        @pl.when(i + NBUF < K)
        def _(b=b, i=i):
          pltpu.async_copy(h_hbm.at[src_v.at[i + NBUF]], rows_v.at[b],
                           gsem.at[b])
      return carry

    lax.fori_loop(0, K // NBUF, group, 0)
    plsc.subcore_barrier()

    pltpu.sync_copy(acc.at[pl.ds(row0, ROWS_PER_SUB)],
                    out_hbm.at[c, pl.ds(row0, ROWS_PER_SUB)])

  return sc_spmm


_BLK = 1000


def _mm1_body(x_ref, w_ref, o_ref):
  o_ref[...] = jnp.dot(x_ref[...], w_ref[...],
                       preferred_element_type=jnp.float32)


def _tc_mm1(x, w1):
  return pl.pallas_call(
      _mm1_body,
      grid=(N_NODES // _BLK,),
      in_specs=[
          pl.BlockSpec((_BLK, D_FEAT), lambda i: (i, 0)),
          pl.BlockSpec((D_FEAT, D_HID), lambda i: (0, 0)),
      ],
      out_specs=pl.BlockSpec((_BLK, D_HID), lambda i: (i, 0)),
      out_shape=jax.ShapeDtypeStruct((N_NODES, D_HID), jnp.float32),
  )(x, w1)


def _mid_body(a_ref, b_ref, b1_ref, w2_ref, o_ref):
  h = jnp.maximum(a_ref[...] + b_ref[...] + b1_ref[...], 0.0)
  o_ref[...] = jnp.dot(h, w2_ref[...], preferred_element_type=jnp.float32)


def _tc_mid(p0, p1, b1row, w2p):
  return pl.pallas_call(
      _mid_body,
      grid=(N_NODES // _BLK,),
      in_specs=[
          pl.BlockSpec((_BLK, D_HID), lambda i: (i, 0)),
          pl.BlockSpec((_BLK, D_HID), lambda i: (i, 0)),
          pl.BlockSpec((1, D_HID), lambda i: (0, 0)),
          pl.BlockSpec((D_HID, D_HID), lambda i: (0, 0)),
      ],
      out_specs=pl.BlockSpec((_BLK, D_HID), lambda i: (i, 0)),
      out_shape=jax.ShapeDtypeStruct((N_NODES, D_HID), jnp.float32),
  )(p0, p1, b1row, w2p)


def _sm_body(a_ref, b_ref, b2_ref, o_ref):
  z = a_ref[...] + b_ref[...] + b2_ref[...]
  col = lax.broadcasted_iota(jnp.int32, z.shape, 1)
  mask = col < N_CLASSES
  zm = jnp.where(mask, z, -jnp.inf)
  m = jnp.max(zm, axis=1, keepdims=True)
  e = jnp.where(mask, jnp.exp(z - m), 0.0)
  ssum = jnp.sum(e, axis=1, keepdims=True)
  o_ref[...] = (z - m) - jnp.log(ssum)


def _tc_softmax(p0, p1, b2row):
  return pl.pallas_call(
      _sm_body,
      grid=(N_NODES // _BLK,),
      in_specs=[
          pl.BlockSpec((_BLK, D_HID), lambda i: (i, 0)),
          pl.BlockSpec((_BLK, D_HID), lambda i: (i, 0)),
          pl.BlockSpec((1, D_HID), lambda i: (0, 0)),
      ],
      out_specs=pl.BlockSpec((_BLK, D_HID), lambda i: (i, 0)),
      out_shape=jax.ShapeDtypeStruct((N_NODES, D_HID), jnp.float32),
  )(p0, p1, b2row)


def kernel(x, edge_index, edge_weight, W1, b1, W2, b2):
  E = edge_index.shape[1]
  K = -(-E // (NW * CHUNK))
  K = -(-K // NBUF) * NBUF
  e_pad = NW * K * CHUNK - E

  src = edge_index[0].astype(jnp.int32)
  dst = edge_index[1].astype(jnp.int32)
  ew = edge_weight.astype(jnp.float32)
  src3 = jnp.pad(src, (0, e_pad)).reshape(NW, K, CHUNK)
  dst3 = jnp.pad(dst, (0, e_pad)).reshape(NW, K, CHUNK)
  ew3 = jnp.pad(ew, (0, e_pad)).reshape(NW, K * CHUNK)
  zeros = jnp.zeros((N_PAD, D_HID), jnp.float32)

  sc_spmm = _make_sc_spmm(K)

  h1 = _tc_mm1(x, W1)
  p1 = sc_spmm(h1, src3, dst3, ew3, zeros)
  w2p = jnp.zeros((D_HID, D_HID), jnp.float32).at[:, :N_CLASSES].set(W2)
  h2 = _tc_mid(p1[0, :N_NODES], p1[1, :N_NODES], b1.reshape(1, D_HID), w2p)
  p2 = sc_spmm(h2, src3, dst3, ew3, zeros)
  b2row = jnp.zeros((1, D_HID), jnp.float32).at[0, :N_CLASSES].set(b2)
  out16 = _tc_softmax(p2[0, :N_NODES], p2[1, :N_NODES], b2row)
  return out16[:, :N_CLASSES]


# speedup vs baseline: 13.4507x; 1.0788x over previous
import functools

import jax
import jax.numpy as jnp
from jax import lax
from jax.experimental import pallas as pl
from jax.experimental.pallas import tpu as pltpu
from jax.experimental.pallas import tpu_sc as plsc

N_NODES = 10000
D_FEAT = 128
D_HID = 16
N_CLASSES = 7

NC = 2
NS = 16
L = 16
NW = NC * NS
CHUNK = 128
NBUF = 4
N_PAD = 10240
ROWS_PER_SUB = N_PAD // NS


_GATHER_DNUMS = lax.GatherDimensionNumbers(
    offset_dims=(), collapsed_slice_dims=(0,), start_index_map=(0,))


def _lane_bcast(v, lane):
  idx = jnp.full((L, 1), lane, jnp.int32)
  return lax.gather(v, idx, _GATHER_DNUMS, slice_sizes=(1,),
                    mode=lax.GatherScatterMode.PROMISE_IN_BOUNDS)


def _make_sc_spmm(K):
  mesh = plsc.VectorSubcoreMesh(core_axis_name="c", subcore_axis_name="s")

  @functools.partial(
      pl.kernel,
      out_type=jax.ShapeDtypeStruct((NC, N_PAD, D_HID), jnp.float32),
      mesh=mesh,
      scratch_types=[
          pltpu.VMEM((K, CHUNK), jnp.int32),
          pltpu.VMEM((K, CHUNK), jnp.int32),
          pltpu.VMEM((K * CHUNK,), jnp.float32),
          pltpu.VMEM((NBUF, CHUNK, D_HID), jnp.float32),
          pltpu.VMEM((NBUF, CHUNK, D_HID), jnp.float32),
          pltpu.VMEM((ROWS_PER_SUB, D_HID), jnp.float32),
          pltpu.VMEM_SHARED((N_PAD, D_HID), jnp.float32),
          pltpu.SemaphoreType.DMA((NBUF,)),
          pltpu.SemaphoreType.DMA,
      ],
      compiler_params=pltpu.CompilerParams(needs_layout_passes=False,
                                           use_tc_tiling_on_sc=False),
  )
  def sc_spmm(h_hbm, src_hbm, dst_hbm, ew_hbm, zeros_hbm, out_hbm,
              src_v, dst_v, ew_v, rows_v, srows_v, zbuf, acc, gsem, ssem):
    c = lax.axis_index("c")
    s = lax.axis_index("s")
    wid = c * NS + s

    pltpu.sync_copy(src_hbm.at[wid], src_v)
    pltpu.sync_copy(dst_hbm.at[wid], dst_v)
    pltpu.sync_copy(ew_hbm.at[wid], ew_v)

    row0 = s * ROWS_PER_SUB
    pltpu.sync_copy(zeros_hbm.at[pl.ds(row0, ROWS_PER_SUB)], zbuf)
    pltpu.sync_copy(zbuf, acc.at[pl.ds(row0, ROWS_PER_SUB)])
    plsc.subcore_barrier()

    for b in range(NBUF):
      pltpu.async_copy(h_hbm.at[src_v.at[b]], rows_v.at[b], gsem.at[b])

    def group(g, carry):
      for b in range(NBUF):
        i = g * NBUF + b
        pltpu.make_async_copy(
            h_hbm.at[src_v.at[i]], rows_v.at[b], gsem.at[b]).wait()

        @pl.when(i >= NBUF)
        def _(b=b, i=i):
          pltpu.make_async_copy(srows_v.at[b], acc.at[dst_v.at[i]],
                                ssem).wait()

        ibase = i * CHUNK

        def qbody(q, carry2, b=b, ibase=ibase):
          wv = ew_v[pl.ds(ibase + q * L, L)]
          for l in range(L):
            wb = _lane_bcast(wv, l)
            j = q * L + l
            srows_v[b, j, :] = rows_v[b, j, :] * wb
          return carry2

        lax.fori_loop(0, CHUNK // L, qbody, 0)
        pltpu.async_copy(srows_v.at[b], acc.at[dst_v.at[i]], ssem, add=True)

        @pl.when(i + NBUF < K)
        def _(b=b, i=i):
          pltpu.async_copy(h_hbm.at[src_v.at[i + NBUF]], rows_v.at[b],
                           gsem.at[b])
      return carry

    lax.fori_loop(0, K // NBUF, group, 0)
    for b in range(NBUF):
      pltpu.make_async_copy(srows_v.at[b], acc.at[dst_v.at[K - NBUF + b]],
                            ssem).wait()
    plsc.subcore_barrier()

    pltpu.sync_copy(acc.at[pl.ds(row0, ROWS_PER_SUB)],
                    out_hbm.at[c, pl.ds(row0, ROWS_PER_SUB)])

  return sc_spmm


_BLK = 1000


def _mm1_body(x_ref, w_ref, o_ref):
  o_ref[...] = jnp.dot(x_ref[...], w_ref[...],
                       preferred_element_type=jnp.float32)


def _tc_mm1(x, w1):
  return pl.pallas_call(
      _mm1_body,
      grid=(N_NODES // _BLK,),
      in_specs=[
          pl.BlockSpec((_BLK, D_FEAT), lambda i: (i, 0)),
          pl.BlockSpec((D_FEAT, D_HID), lambda i: (0, 0)),
      ],
      out_specs=pl.BlockSpec((_BLK, D_HID), lambda i: (i, 0)),
      out_shape=jax.ShapeDtypeStruct((N_NODES, D_HID), jnp.float32),
  )(x, w1)


def _mid_body(a_ref, b_ref, b1_ref, w2_ref, o_ref):
  h = jnp.maximum(a_ref[...] + b_ref[...] + b1_ref[...], 0.0)
  o_ref[...] = jnp.dot(h, w2_ref[...], preferred_element_type=jnp.float32)


def _tc_mid(p0, p1, b1row, w2p):
  return pl.pallas_call(
      _mid_body,
      grid=(N_NODES // _BLK,),
      in_specs=[
          pl.BlockSpec((_BLK, D_HID), lambda i: (i, 0)),
          pl.BlockSpec((_BLK, D_HID), lambda i: (i, 0)),
          pl.BlockSpec((1, D_HID), lambda i: (0, 0)),
          pl.BlockSpec((D_HID, D_HID), lambda i: (0, 0)),
      ],
      out_specs=pl.BlockSpec((_BLK, D_HID), lambda i: (i, 0)),
      out_shape=jax.ShapeDtypeStruct((N_NODES, D_HID), jnp.float32),
  )(p0, p1, b1row, w2p)


def _sm_body(a_ref, b_ref, b2_ref, o_ref):
  z = a_ref[...] + b_ref[...] + b2_ref[...]
  col = lax.broadcasted_iota(jnp.int32, z.shape, 1)
  mask = col < N_CLASSES
  zm = jnp.where(mask, z, -jnp.inf)
  m = jnp.max(zm, axis=1, keepdims=True)
  e = jnp.where(mask, jnp.exp(z - m), 0.0)
  ssum = jnp.sum(e, axis=1, keepdims=True)
  o_ref[...] = (z - m) - jnp.log(ssum)


def _tc_softmax(p0, p1, b2row):
  return pl.pallas_call(
      _sm_body,
      grid=(N_NODES // _BLK,),
      in_specs=[
          pl.BlockSpec((_BLK, D_HID), lambda i: (i, 0)),
          pl.BlockSpec((_BLK, D_HID), lambda i: (i, 0)),
          pl.BlockSpec((1, D_HID), lambda i: (0, 0)),
      ],
      out_specs=pl.BlockSpec((_BLK, D_HID), lambda i: (i, 0)),
      out_shape=jax.ShapeDtypeStruct((N_NODES, D_HID), jnp.float32),
  )(p0, p1, b2row)


def kernel(x, edge_index, edge_weight, W1, b1, W2, b2):
  E = edge_index.shape[1]
  K = -(-E // (NW * CHUNK))
  K = -(-K // NBUF) * NBUF
  e_pad = NW * K * CHUNK - E

  src = edge_index[0].astype(jnp.int32)
  dst = edge_index[1].astype(jnp.int32)
  ew = edge_weight.astype(jnp.float32)
  src3 = jnp.pad(src, (0, e_pad)).reshape(NW, K, CHUNK)
  dst3 = jnp.pad(dst, (0, e_pad)).reshape(NW, K, CHUNK)
  ew3 = jnp.pad(ew, (0, e_pad)).reshape(NW, K * CHUNK)
  zeros = jnp.zeros((N_PAD, D_HID), jnp.float32)

  sc_spmm = _make_sc_spmm(K)

  h1 = _tc_mm1(x, W1)
  p1 = sc_spmm(h1, src3, dst3, ew3, zeros)
  w2p = jnp.zeros((D_HID, D_HID), jnp.float32).at[:, :N_CLASSES].set(W2)
  h2 = _tc_mid(p1[0, :N_NODES], p1[1, :N_NODES], b1.reshape(1, D_HID), w2p)
  p2 = sc_spmm(h2, src3, dst3, ew3, zeros)
  b2row = jnp.zeros((1, D_HID), jnp.float32).at[0, :N_CLASSES].set(b2)
  out16 = _tc_softmax(p2[0, :N_NODES], p2[1, :N_NODES], b2row)
  return out16[:, :N_CLASSES]


# speedup vs baseline: 14.7782x; 1.0987x over previous
import functools

import jax
import jax.numpy as jnp
from jax import lax
from jax.experimental import pallas as pl
from jax.experimental.pallas import tpu as pltpu
from jax.experimental.pallas import tpu_sc as plsc

N_NODES = 10000
D_FEAT = 128
D_HID = 16
N_CLASSES = 7

NC = 2
NS = 16
L = 16
CHUNK = 128
NBUF = 4
N_PAD = 10240
ROWS_PER_SUB = N_PAD // NS
HALF = N_PAD // NC
ROWS_PER_SUB_OUT = HALF // NS


_GATHER_DNUMS = lax.GatherDimensionNumbers(
    offset_dims=(), collapsed_slice_dims=(0,), start_index_map=(0,))


def _lane_bcast(v, lane):
  idx = jnp.full((L, 1), lane, jnp.int32)
  return lax.gather(v, idx, _GATHER_DNUMS, slice_sizes=(1,),
                    mode=lax.GatherScatterMode.PROMISE_IN_BOUNDS)


def _make_sc_2layer(K):
  mesh = plsc.VectorSubcoreMesh(core_axis_name="c", subcore_axis_name="s")

  @functools.partial(
      pl.kernel,
      out_type=jax.ShapeDtypeStruct((N_PAD, D_HID), jnp.float32),
      mesh=mesh,
      scratch_types=[
          pltpu.VMEM((K, CHUNK), jnp.int32),
          pltpu.VMEM((K, CHUNK), jnp.int32),
          pltpu.VMEM((K * CHUNK,), jnp.float32),
          pltpu.VMEM((NBUF, CHUNK, D_HID), jnp.float32),
          pltpu.VMEM((NBUF, CHUNK, D_HID), jnp.float32),
          pltpu.VMEM((ROWS_PER_SUB, D_HID), jnp.float32),
          pltpu.VMEM((L,), jnp.float32),
          pltpu.VMEM_SHARED((N_PAD, D_HID), jnp.float32),
          pltpu.VMEM_SHARED((N_PAD, D_HID), jnp.float32),
          pltpu.SemaphoreType.DMA((NBUF,)),
          pltpu.SemaphoreType.DMA,
      ],
      compiler_params=pltpu.CompilerParams(needs_layout_passes=False,
                                           use_tc_tiling_on_sc=False),
  )
  def sc_2layer(h_hbm, src_hbm, dst_hbm, ew_hbm, zeros_hbm, b1_hbm, out_hbm,
                src_v, dst_v, ew_v, rows_v, srows_v, zbuf, b1_v,
                acc1, acc2, gsem, ssem):
    c = lax.axis_index("c")
    s = lax.axis_index("s")

    pltpu.sync_copy(src_hbm.at[s], src_v)
    pltpu.sync_copy(dst_hbm.at[s], dst_v)
    pltpu.sync_copy(ew_hbm.at[s], ew_v)
    pltpu.sync_copy(b1_hbm, b1_v)

    row0 = s * ROWS_PER_SUB
    pltpu.sync_copy(zeros_hbm.at[pl.ds(row0, ROWS_PER_SUB)], zbuf)
    pltpu.sync_copy(zbuf, acc1.at[pl.ds(row0, ROWS_PER_SUB)])
    pltpu.sync_copy(zbuf, acc2.at[pl.ds(row0, ROWS_PER_SUB)])
    plsc.subcore_barrier()

    b1vec = b1_v[...]

    def run_layer(table, acc, fixup):
      for b in range(NBUF):
        pltpu.async_copy(table.at[src_v.at[b]], rows_v.at[b], gsem.at[b])

      def group(g, carry):
        for b in range(NBUF):
          i = g * NBUF + b
          pltpu.make_async_copy(
              table.at[src_v.at[i]], rows_v.at[b], gsem.at[b]).wait()

          @pl.when(i >= NBUF)
          def _(b=b, i=i):
            pltpu.make_async_copy(srows_v.at[b], acc.at[dst_v.at[i]],
                                  ssem).wait()

          ibase = i * CHUNK

          def qbody(q, carry2, b=b, ibase=ibase):
            wv = ew_v[pl.ds(ibase + q * L, L)]
            for l in range(L):
              wb = _lane_bcast(wv, l)
              j = q * L + l
              srows_v[b, j, :] = fixup(rows_v[b, j, :]) * wb
            return carry2

          lax.fori_loop(0, CHUNK // L, qbody, 0)
          pltpu.async_copy(srows_v.at[b], acc.at[dst_v.at[i]], ssem,
                           add=True)

          @pl.when(i + NBUF < K)
          def _(b=b, i=i):
            pltpu.async_copy(table.at[src_v.at[i + NBUF]], rows_v.at[b],
                             gsem.at[b])
        return carry

      lax.fori_loop(0, K // NBUF, group, 0)
      for b in range(NBUF):
        pltpu.make_async_copy(srows_v.at[b], acc.at[dst_v.at[K - NBUF + b]],
                              ssem).wait()
      plsc.subcore_barrier()

    run_layer(h_hbm, acc1, lambda r: r)
    run_layer(acc1, acc2, lambda r: jnp.maximum(r + b1vec, 0.0))

    out0 = c * HALF + s * ROWS_PER_SUB_OUT
    pltpu.sync_copy(acc2.at[pl.ds(out0, ROWS_PER_SUB_OUT)],
                    out_hbm.at[pl.ds(out0, ROWS_PER_SUB_OUT)])

  return sc_2layer


_BLK = 1000


def _mm1_body(x_ref, w_ref, o_ref):
  o_ref[...] = jnp.dot(x_ref[...], w_ref[...],
                       preferred_element_type=jnp.float32)


def _tc_mm1(x, w1):
  return pl.pallas_call(
      _mm1_body,
      grid=(N_NODES // _BLK,),
      in_specs=[
          pl.BlockSpec((_BLK, D_FEAT), lambda i: (i, 0)),
          pl.BlockSpec((D_FEAT, D_HID), lambda i: (0, 0)),
      ],
      out_specs=pl.BlockSpec((_BLK, D_HID), lambda i: (i, 0)),
      out_shape=jax.ShapeDtypeStruct((N_NODES, D_HID), jnp.float32),
  )(x, w1)


def _sm_body(q_ref, w2_ref, b2_ref, o_ref):
  z = jnp.dot(q_ref[...], w2_ref[...],
              preferred_element_type=jnp.float32) + b2_ref[...]
  col = lax.broadcasted_iota(jnp.int32, z.shape, 1)
  mask = col < N_CLASSES
  zm = jnp.where(mask, z, -jnp.inf)
  m = jnp.max(zm, axis=1, keepdims=True)
  e = jnp.where(mask, jnp.exp(z - m), 0.0)
  ssum = jnp.sum(e, axis=1, keepdims=True)
  o_ref[...] = (z - m) - jnp.log(ssum)


def _tc_final(q, w2p, b2row):
  return pl.pallas_call(
      _sm_body,
      grid=(N_NODES // _BLK,),
      in_specs=[
          pl.BlockSpec((_BLK, D_HID), lambda i: (i, 0)),
          pl.BlockSpec((D_HID, D_HID), lambda i: (0, 0)),
          pl.BlockSpec((1, D_HID), lambda i: (0, 0)),
      ],
      out_specs=pl.BlockSpec((_BLK, D_HID), lambda i: (i, 0)),
      out_shape=jax.ShapeDtypeStruct((N_NODES, D_HID), jnp.float32),
  )(q, w2p, b2row)


def kernel(x, edge_index, edge_weight, W1, b1, W2, b2):
  E = edge_index.shape[1]
  K = -(-E // (NS * CHUNK))
  K = -(-K // NBUF) * NBUF
  e_pad = NS * K * CHUNK - E

  src = edge_index[0].astype(jnp.int32)
  dst = edge_index[1].astype(jnp.int32)
  ew = edge_weight.astype(jnp.float32)
  src3 = jnp.pad(src, (0, e_pad)).reshape(NS, K, CHUNK)
  dst3 = jnp.pad(dst, (0, e_pad)).reshape(NS, K, CHUNK)
  ewf = jnp.pad(ew, (0, e_pad)).reshape(NS, K * CHUNK)
  zeros = jnp.zeros((N_PAD, D_HID), jnp.float32)

  h1 = _tc_mm1(x, W1)
  q = _make_sc_2layer(K)(h1, src3, dst3, ewf, zeros, b1)
  w2p = jnp.zeros((D_HID, D_HID), jnp.float32).at[:, :N_CLASSES].set(W2)
  b2row = jnp.zeros((1, D_HID), jnp.float32).at[0, :N_CLASSES].set(b2)
  out16 = _tc_final(q[:N_NODES], w2p, b2row)
  return out16[:, :N_CLASSES]


# speedup vs baseline: 18.6864x; 1.2645x over previous
import functools

import jax
import jax.numpy as jnp
from jax import lax
from jax.experimental import pallas as pl
from jax.experimental.pallas import tpu as pltpu
from jax.experimental.pallas import tpu_sc as plsc

N_NODES = 10000
D_FEAT = 128
D_HID = 16
N_CLASSES = 7

NC = 2
NS = 16
L = 16
NW = NC * NS
CHUNK = 128
NBUF = 4
N_PAD = 10240
ROWS_PER_SUB = N_PAD // NS


_GATHER_DNUMS = lax.GatherDimensionNumbers(
    offset_dims=(), collapsed_slice_dims=(0,), start_index_map=(0,))


def _lane_bcast(v, lane):
  idx = jnp.full((L, 1), lane, jnp.int32)
  return lax.gather(v, idx, _GATHER_DNUMS, slice_sizes=(1,),
                    mode=lax.GatherScatterMode.PROMISE_IN_BOUNDS)


def _make_sc_2layer(K):
  mesh = plsc.VectorSubcoreMesh(core_axis_name="c", subcore_axis_name="s")

  @functools.partial(
      pl.kernel,
      out_type=(jax.ShapeDtypeStruct((NC, N_PAD, D_HID), jnp.float32),
                jax.ShapeDtypeStruct((NC, N_PAD, D_HID), jnp.float32)),
      mesh=mesh,
      scratch_types=[
          pltpu.VMEM((K, CHUNK), jnp.int32),
          pltpu.VMEM((K, CHUNK), jnp.int32),
          pltpu.VMEM((K * CHUNK,), jnp.float32),
          pltpu.VMEM((NBUF, CHUNK, D_HID), jnp.float32),
          pltpu.VMEM((NBUF, CHUNK, D_HID), jnp.float32),
          pltpu.VMEM((ROWS_PER_SUB, D_HID), jnp.float32),
          pltpu.VMEM((ROWS_PER_SUB, D_HID), jnp.float32),
          pltpu.VMEM((L,), jnp.float32),
          pltpu.VMEM_SHARED((N_PAD, D_HID), jnp.float32),
          pltpu.VMEM_SHARED((N_PAD, D_HID), jnp.float32),
          pltpu.SemaphoreType.DMA((NBUF,)),
          pltpu.SemaphoreType.DMA,
          pltpu.SemaphoreType.REGULAR,
      ],
      compiler_params=pltpu.CompilerParams(needs_layout_passes=False,
                                           use_tc_tiling_on_sc=False),
  )
  def sc_2layer(h_hbm, src_hbm, dst_hbm, ew_hbm, zeros_hbm, b1_hbm,
                out_hbm, p1_hbm,
                src_v, dst_v, ew_v, rows_v, srows_v, zbuf, pbuf, b1_v,
                acc1, acc2, gsem, ssem, xsem):
    c = lax.axis_index("c")
    s = lax.axis_index("s")
    wid = c * NS + s

    pltpu.sync_copy(src_hbm.at[wid], src_v)
    pltpu.sync_copy(dst_hbm.at[wid], dst_v)
    pltpu.sync_copy(ew_hbm.at[wid], ew_v)
    pltpu.sync_copy(b1_hbm, b1_v)

    row0 = s * ROWS_PER_SUB
    pltpu.sync_copy(zeros_hbm.at[pl.ds(row0, ROWS_PER_SUB)], zbuf)
    pltpu.sync_copy(zbuf, acc1.at[pl.ds(row0, ROWS_PER_SUB)])
    pltpu.sync_copy(zbuf, acc2.at[pl.ds(row0, ROWS_PER_SUB)])
    plsc.subcore_barrier()

    b1vec = b1_v[...]

    def run_layer(table, acc, fixup):
      for b in range(NBUF):
        pltpu.async_copy(table.at[src_v.at[b]], rows_v.at[b], gsem.at[b])

      def group(g, carry):
        for b in range(NBUF):
          i = g * NBUF + b
          pltpu.make_async_copy(
              table.at[src_v.at[i]], rows_v.at[b], gsem.at[b]).wait()

          @pl.when(i >= NBUF)
          def _(b=b, i=i):
            pltpu.make_async_copy(srows_v.at[b], acc.at[dst_v.at[i]],
                                  ssem).wait()

          ibase = i * CHUNK

          def qbody(q, carry2, b=b, ibase=ibase):
            wv = ew_v[pl.ds(ibase + q * L, L)]
            for l in range(L):
              wb = _lane_bcast(wv, l)
              j = q * L + l
              srows_v[b, j, :] = fixup(rows_v[b, j, :]) * wb
            return carry2

          lax.fori_loop(0, CHUNK // L, qbody, 0)
          pltpu.async_copy(srows_v.at[b], acc.at[dst_v.at[i]], ssem,
                           add=True)

          @pl.when(i + NBUF < K)
          def _(b=b, i=i):
            pltpu.async_copy(table.at[src_v.at[i + NBUF]], rows_v.at[b],
                             gsem.at[b])
        return carry

      lax.fori_loop(0, K // NBUF, group, 0)
      for b in range(NBUF):
        pltpu.make_async_copy(srows_v.at[b], acc.at[dst_v.at[K - NBUF + b]],
                              ssem).wait()
      plsc.subcore_barrier()

    run_layer(h_hbm, acc1, lambda r: r)

    pltpu.sync_copy(acc1.at[pl.ds(row0, ROWS_PER_SUB)],
                    p1_hbm.at[c, pl.ds(row0, ROWS_PER_SUB)])
    plsc.subcore_barrier()
    pltpu.core_barrier(xsem, core_axis_name="c")
    pltpu.sync_copy(p1_hbm.at[1 - c, pl.ds(row0, ROWS_PER_SUB)], pbuf)
    pltpu.sync_copy(acc1.at[pl.ds(row0, ROWS_PER_SUB)], zbuf)

    def addrow(j, carry):
      zbuf[j, :] = zbuf[j, :] + pbuf[j, :]
      return carry

    lax.fori_loop(0, ROWS_PER_SUB, addrow, 0, unroll=8)
    pltpu.sync_copy(zbuf, acc1.at[pl.ds(row0, ROWS_PER_SUB)])
    plsc.subcore_barrier()

    run_layer(acc1, acc2, lambda r: jnp.maximum(r + b1vec, 0.0))

    pltpu.sync_copy(acc2.at[pl.ds(row0, ROWS_PER_SUB)],
                    out_hbm.at[c, pl.ds(row0, ROWS_PER_SUB)])

  return sc_2layer


_BLK = 1000


def _mm1_body(x_ref, w_ref, o_ref):
  o_ref[...] = jnp.dot(x_ref[...], w_ref[...],
                       preferred_element_type=jnp.float32)


def _tc_mm1(x, w1):
  return pl.pallas_call(
      _mm1_body,
      grid=(N_NODES // _BLK,),
      in_specs=[
          pl.BlockSpec((_BLK, D_FEAT), lambda i: (i, 0)),
          pl.BlockSpec((D_FEAT, D_HID), lambda i: (0, 0)),
      ],
      out_specs=pl.BlockSpec((_BLK, D_HID), lambda i: (i, 0)),
      out_shape=jax.ShapeDtypeStruct((N_NODES, D_HID), jnp.float32),
  )(x, w1)


def _sm_body(q_ref, w2_ref, b2_ref, o_ref):
  z = jnp.dot(q_ref[0] + q_ref[1], w2_ref[...],
              preferred_element_type=jnp.float32) + b2_ref[...]
  col = lax.broadcasted_iota(jnp.int32, z.shape, 1)
  mask = col < N_CLASSES
  zm = jnp.where(mask, z, -jnp.inf)
  m = jnp.max(zm, axis=1, keepdims=True)
  e = jnp.where(mask, jnp.exp(z - m), 0.0)
  ssum = jnp.sum(e, axis=1, keepdims=True)
  o_ref[...] = (z - m) - jnp.log(ssum)


def _tc_final(qp, w2p, b2row):
  return pl.pallas_call(
      _sm_body,
      grid=(N_NODES // _BLK,),
      in_specs=[
          pl.BlockSpec((NC, _BLK, D_HID), lambda i: (0, i, 0)),
          pl.BlockSpec((D_HID, D_HID), lambda i: (0, 0)),
          pl.BlockSpec((1, D_HID), lambda i: (0, 0)),
      ],
      out_specs=pl.BlockSpec((_BLK, D_HID), lambda i: (i, 0)),
      out_shape=jax.ShapeDtypeStruct((N_NODES, D_HID), jnp.float32),
  )(qp, w2p, b2row)


def kernel(x, edge_index, edge_weight, W1, b1, W2, b2):
  E = edge_index.shape[1]
  K = -(-E // (NW * CHUNK))
  K = -(-K // NBUF) * NBUF
  e_pad = NW * K * CHUNK - E

  src = edge_index[0].astype(jnp.int32)
  dst = edge_index[1].astype(jnp.int32)
  ew = edge_weight.astype(jnp.float32)
  src3 = jnp.pad(src, (0, e_pad)).reshape(NW, K, CHUNK)
  dst3 = jnp.pad(dst, (0, e_pad)).reshape(NW, K, CHUNK)
  ewf = jnp.pad(ew, (0, e_pad)).reshape(NW, K * CHUNK)
  zeros = jnp.zeros((N_PAD, D_HID), jnp.float32)

  h1 = _tc_mm1(x, W1)
  qp, _ = _make_sc_2layer(K)(h1, src3, dst3, ewf, zeros, b1)
  w2p = jnp.zeros((D_HID, D_HID), jnp.float32).at[:, :N_CLASSES].set(W2)
  b2row = jnp.zeros((1, D_HID), jnp.float32).at[0, :N_CLASSES].set(b2)
  out16 = _tc_final(qp, w2p, b2row)
  return out16[:, :N_CLASSES]


# speedup vs baseline: 18.7664x; 1.0043x over previous
import functools

import jax
import jax.numpy as jnp
from jax import lax
from jax.experimental import pallas as pl
from jax.experimental.pallas import tpu as pltpu
from jax.experimental.pallas import tpu_sc as plsc

N_NODES = 10000
D_FEAT = 128
D_HID = 16
N_CLASSES = 7

NC = 2
NS = 16
L = 16
NW = NC * NS
CHUNK = 128
NBUF = 4
N_PAD = 10240
ROWS_PER_SUB = N_PAD // NS


_GATHER_DNUMS = lax.GatherDimensionNumbers(
    offset_dims=(), collapsed_slice_dims=(0,), start_index_map=(0,))


def _lane_bcast(v, lane):
  idx = jnp.full((L, 1), lane, jnp.int32)
  return lax.gather(v, idx, _GATHER_DNUMS, slice_sizes=(1,),
                    mode=lax.GatherScatterMode.PROMISE_IN_BOUNDS)


def _make_sc_2layer(K):
  mesh = plsc.VectorSubcoreMesh(core_axis_name="c", subcore_axis_name="s")

  @functools.partial(
      pl.kernel,
      out_type=(jax.ShapeDtypeStruct((NC, N_PAD, D_HID), jnp.float32),
                jax.ShapeDtypeStruct((NC, N_PAD, D_HID), jnp.float32)),
      mesh=mesh,
      scratch_types=[
          pltpu.VMEM((K, CHUNK), jnp.int32),
          pltpu.VMEM((K, CHUNK), jnp.int32),
          pltpu.VMEM((K * CHUNK,), jnp.float32),
          pltpu.VMEM((NBUF, CHUNK, D_HID), jnp.float32),
          pltpu.VMEM((NBUF, CHUNK, D_HID), jnp.float32),
          pltpu.VMEM((ROWS_PER_SUB, D_HID), jnp.float32),
          pltpu.VMEM((ROWS_PER_SUB, D_HID), jnp.float32),
          pltpu.VMEM((L,), jnp.float32),
          pltpu.VMEM_SHARED((N_PAD, D_HID), jnp.float32),
          pltpu.VMEM_SHARED((N_PAD, D_HID), jnp.float32),
          pltpu.SemaphoreType.DMA((NBUF,)),
          pltpu.SemaphoreType.DMA,
          pltpu.SemaphoreType.REGULAR,
      ],
      compiler_params=pltpu.CompilerParams(needs_layout_passes=False,
                                           use_tc_tiling_on_sc=False),
  )
  def sc_2layer(h_hbm, src_hbm, dst_hbm, ew_hbm, zeros_hbm, b1_hbm,
                out_hbm, p1_hbm,
                src_v, dst_v, ew_v, rows_v, srows_v, zbuf, pbuf, b1_v,
                acc1, acc2, gsem, ssem, xsem):
    c = lax.axis_index("c")
    s = lax.axis_index("s")
    wid = c * NS + s

    pltpu.sync_copy(src_hbm.at[wid], src_v)
    pltpu.sync_copy(dst_hbm.at[wid], dst_v)
    pltpu.sync_copy(ew_hbm.at[wid], ew_v)
    pltpu.sync_copy(b1_hbm, b1_v)

    row0 = s * ROWS_PER_SUB
    pltpu.sync_copy(zeros_hbm.at[pl.ds(row0, ROWS_PER_SUB)], zbuf)
    pltpu.sync_copy(zbuf, acc1.at[pl.ds(row0, ROWS_PER_SUB)])
    pltpu.sync_copy(zbuf, acc2.at[pl.ds(row0, ROWS_PER_SUB)])
    plsc.subcore_barrier()

    b1vec = b1_v[...]

    def run_layer(table, acc, fixup):
      for b in range(NBUF):
        pltpu.async_copy(table.at[src_v.at[b]], rows_v.at[b], gsem.at[b])

      def group(g, carry):
        for b in range(NBUF):
          i = g * NBUF + b
          pltpu.make_async_copy(
              table.at[src_v.at[i]], rows_v.at[b], gsem.at[b]).wait()

          @pl.when(i >= NBUF)
          def _(b=b, i=i):
            pltpu.make_async_copy(srows_v.at[b], acc.at[dst_v.at[i]],
                                  ssem).wait()

          ibase = i * CHUNK

          @plsc.parallel_loop(0, CHUNK // L, unroll=2)
          def _(q, b=b, ibase=ibase):
            wv = ew_v[pl.ds(ibase + q * L, L)]
            for l in range(L):
              wb = _lane_bcast(wv, l)
              j = q * L + l
              srows_v[b, j, :] = fixup(rows_v[b, j, :]) * wb
          pltpu.async_copy(srows_v.at[b], acc.at[dst_v.at[i]], ssem,
                           add=True)

          @pl.when(i + NBUF < K)
          def _(b=b, i=i):
            pltpu.async_copy(table.at[src_v.at[i + NBUF]], rows_v.at[b],
                             gsem.at[b])
        return carry

      lax.fori_loop(0, K // NBUF, group, 0)
      for b in range(NBUF):
        pltpu.make_async_copy(srows_v.at[b], acc.at[dst_v.at[K - NBUF + b]],
                              ssem).wait()
      plsc.subcore_barrier()

    run_layer(h_hbm, acc1, lambda r: r)

    pltpu.sync_copy(acc1.at[pl.ds(row0, ROWS_PER_SUB)],
                    p1_hbm.at[c, pl.ds(row0, ROWS_PER_SUB)])
    plsc.subcore_barrier()
    pltpu.core_barrier(xsem, core_axis_name="c")
    pltpu.sync_copy(p1_hbm.at[1 - c, pl.ds(row0, ROWS_PER_SUB)], pbuf)
    pltpu.sync_copy(acc1.at[pl.ds(row0, ROWS_PER_SUB)], zbuf)

    @plsc.parallel_loop(0, ROWS_PER_SUB, unroll=8)
    def _(j):
      zbuf[j, :] = zbuf[j, :] + pbuf[j, :]
    pltpu.sync_copy(zbuf, acc1.at[pl.ds(row0, ROWS_PER_SUB)])
    plsc.subcore_barrier()

    run_layer(acc1, acc2, lambda r: jnp.maximum(r + b1vec, 0.0))

    pltpu.sync_copy(acc2.at[pl.ds(row0, ROWS_PER_SUB)],
                    out_hbm.at[c, pl.ds(row0, ROWS_PER_SUB)])

  return sc_2layer


_BLK = 1000


def _mm1_body(x_ref, w_ref, o_ref):
  o_ref[...] = jnp.dot(x_ref[...], w_ref[...],
                       preferred_element_type=jnp.float32)


def _tc_mm1(x, w1):
  return pl.pallas_call(
      _mm1_body,
      grid=(N_NODES // _BLK,),
      in_specs=[
          pl.BlockSpec((_BLK, D_FEAT), lambda i: (i, 0)),
          pl.BlockSpec((D_FEAT, D_HID), lambda i: (0, 0)),
      ],
      out_specs=pl.BlockSpec((_BLK, D_HID), lambda i: (i, 0)),
      out_shape=jax.ShapeDtypeStruct((N_NODES, D_HID), jnp.float32),
  )(x, w1)


def _sm_body(q_ref, w2_ref, b2_ref, o_ref):
  z = jnp.dot(q_ref[0] + q_ref[1], w2_ref[...],
              preferred_element_type=jnp.float32) + b2_ref[...]
  col = lax.broadcasted_iota(jnp.int32, z.shape, 1)
  mask = col < N_CLASSES
  zm = jnp.where(mask, z, -jnp.inf)
  m = jnp.max(zm, axis=1, keepdims=True)
  e = jnp.where(mask, jnp.exp(z - m), 0.0)
  ssum = jnp.sum(e, axis=1, keepdims=True)
  o_ref[...] = (z - m) - jnp.log(ssum)


def _tc_final(qp, w2p, b2row):
  return pl.pallas_call(
      _sm_body,
      grid=(N_NODES // _BLK,),
      in_specs=[
          pl.BlockSpec((NC, _BLK, D_HID), lambda i: (0, i, 0)),
          pl.BlockSpec((D_HID, D_HID), lambda i: (0, 0)),
          pl.BlockSpec((1, D_HID), lambda i: (0, 0)),
      ],
      out_specs=pl.BlockSpec((_BLK, D_HID), lambda i: (i, 0)),
      out_shape=jax.ShapeDtypeStruct((N_NODES, D_HID), jnp.float32),
  )(qp, w2p, b2row)


def kernel(x, edge_index, edge_weight, W1, b1, W2, b2):
  E = edge_index.shape[1]
  K = -(-E // (NW * CHUNK))
  K = -(-K // NBUF) * NBUF
  e_pad = NW * K * CHUNK - E

  src = edge_index[0].astype(jnp.int32)
  dst = edge_index[1].astype(jnp.int32)
  ew = edge_weight.astype(jnp.float32)
  src3 = jnp.pad(src, (0, e_pad)).reshape(NW, K, CHUNK)
  dst3 = jnp.pad(dst, (0, e_pad)).reshape(NW, K, CHUNK)
  ewf = jnp.pad(ew, (0, e_pad)).reshape(NW, K * CHUNK)
  zeros = jnp.zeros((N_PAD, D_HID), jnp.float32)

  h1 = _tc_mm1(x, W1)
  qp, _ = _make_sc_2layer(K)(h1, src3, dst3, ewf, zeros, b1)
  w2p = jnp.zeros((D_HID, D_HID), jnp.float32).at[:, :N_CLASSES].set(W2)
  b2row = jnp.zeros((1, D_HID), jnp.float32).at[0, :N_CLASSES].set(b2)
  out16 = _tc_final(qp, w2p, b2row)
  return out16[:, :N_CLASSES]


# speedup vs baseline: 19.5001x; 1.0391x over previous
import functools

import jax
import jax.numpy as jnp
from jax import lax
from jax.experimental import pallas as pl
from jax.experimental.pallas import tpu as pltpu
from jax.experimental.pallas import tpu_sc as plsc

N_NODES = 10000
D_FEAT = 128
D_HID = 16
N_CLASSES = 7

NC = 2
NS = 16
L = 16
NW = NC * NS
CHUNK = 128
NBUF = 4
N_PAD = 10240
ROWS_PER_SUB = N_PAD // NS


_GATHER_DNUMS = lax.GatherDimensionNumbers(
    offset_dims=(), collapsed_slice_dims=(0,), start_index_map=(0,))


def _lane_bcast(v, lane):
  idx = jnp.full((L, 1), lane, jnp.int32)
  return lax.gather(v, idx, _GATHER_DNUMS, slice_sizes=(1,),
                    mode=lax.GatherScatterMode.PROMISE_IN_BOUNDS)


def _make_sc_2layer(K):
  mesh = plsc.VectorSubcoreMesh(core_axis_name="c", subcore_axis_name="s")

  @functools.partial(
      pl.kernel,
      out_type=(jax.ShapeDtypeStruct((NC, N_PAD, D_HID), jnp.float32),
                jax.ShapeDtypeStruct((NC, N_PAD, D_HID), jnp.float32)),
      mesh=mesh,
      scratch_types=[
          pltpu.VMEM((K, CHUNK), jnp.int32),
          pltpu.VMEM((K, CHUNK), jnp.int32),
          pltpu.VMEM((K * CHUNK,), jnp.float32),
          pltpu.VMEM((NBUF, CHUNK, D_HID), jnp.float32),
          pltpu.VMEM((NBUF, CHUNK, D_HID), jnp.float32),
          pltpu.VMEM((ROWS_PER_SUB, D_HID), jnp.float32),
          pltpu.VMEM((ROWS_PER_SUB, D_HID), jnp.float32),
          pltpu.VMEM((L,), jnp.float32),
          pltpu.VMEM_SHARED((N_PAD, D_HID), jnp.float32),
          pltpu.VMEM_SHARED((N_PAD, D_HID), jnp.float32),
          pltpu.SemaphoreType.DMA((NBUF,)),
          pltpu.SemaphoreType.DMA,
          pltpu.SemaphoreType.REGULAR,
      ],
      compiler_params=pltpu.CompilerParams(needs_layout_passes=False,
                                           use_tc_tiling_on_sc=False),
  )
  def sc_2layer(h_hbm, src_hbm, dst_hbm, ew_hbm, zeros_hbm, b1_hbm,
                out_hbm, p1_hbm,
                src_v, dst_v, ew_v, rows_v, srows_v, zbuf, pbuf, b1_v,
                acc1, acc2, gsem, ssem, xsem):
    c = lax.axis_index("c")
    s = lax.axis_index("s")
    wid = c * NS + s

    pltpu.sync_copy(src_hbm.at[wid], src_v)
    pltpu.sync_copy(dst_hbm.at[wid], dst_v)
    pltpu.sync_copy(ew_hbm.at[wid], ew_v)
    pltpu.sync_copy(b1_hbm, b1_v)

    row0 = s * ROWS_PER_SUB
    pltpu.sync_copy(zeros_hbm.at[pl.ds(row0, ROWS_PER_SUB)], zbuf)
    pltpu.sync_copy(zbuf, acc1.at[pl.ds(row0, ROWS_PER_SUB)])
    pltpu.sync_copy(zbuf, acc2.at[pl.ds(row0, ROWS_PER_SUB)])
    plsc.subcore_barrier()

    b1vec = b1_v[...]

    def run_layer(table, acc, fixup):
      for b in range(NBUF):
        pltpu.async_copy(table.at[src_v.at[b]], rows_v.at[b], gsem.at[b])

      def group(g, carry):
        for b in range(NBUF):
          i = g * NBUF + b
          pltpu.make_async_copy(
              table.at[src_v.at[i]], rows_v.at[b], gsem.at[b]).wait()

          @pl.when(i >= NBUF)
          def _(b=b, i=i):
            pltpu.make_async_copy(srows_v.at[b], acc.at[dst_v.at[i]],
                                  ssem).wait()

          ibase = i * CHUNK

          @plsc.parallel_loop(0, CHUNK // L, unroll=2)
          def _(q, b=b, ibase=ibase):
            wv = ew_v[pl.ds(ibase + q * L, L)]
            for l in range(L):
              wb = _lane_bcast(wv, l)
              j = q * L + l
              srows_v[b, j, :] = fixup(rows_v[b, j, :]) * wb
          pltpu.async_copy(srows_v.at[b], acc.at[dst_v.at[i]], ssem,
                           add=True)

          @pl.when(i + NBUF < K)
          def _(b=b, i=i):
            pltpu.async_copy(table.at[src_v.at[i + NBUF]], rows_v.at[b],
                             gsem.at[b])
        return carry

      lax.fori_loop(0, K // NBUF, group, 0)
      for b in range(NBUF):
        pltpu.make_async_copy(srows_v.at[b], acc.at[dst_v.at[K - NBUF + b]],
                              ssem).wait()
      plsc.subcore_barrier()

    run_layer(h_hbm, acc1, lambda r: r)

    pltpu.sync_copy(acc1.at[pl.ds(row0, ROWS_PER_SUB)],
                    p1_hbm.at[c, pl.ds(row0, ROWS_PER_SUB)])
    plsc.subcore_barrier()
    pltpu.core_barrier(xsem, core_axis_name="c")
    pltpu.sync_copy(p1_hbm.at[1 - c, pl.ds(row0, ROWS_PER_SUB)], pbuf)
    pltpu.sync_copy(acc1.at[pl.ds(row0, ROWS_PER_SUB)], zbuf)

    @plsc.parallel_loop(0, ROWS_PER_SUB, unroll=8)
    def _(j):
      zbuf[j, :] = zbuf[j, :] + pbuf[j, :]
    pltpu.sync_copy(zbuf, acc1.at[pl.ds(row0, ROWS_PER_SUB)])
    plsc.subcore_barrier()

    run_layer(acc1, acc2, lambda r: jnp.maximum(r + b1vec, 0.0))

    pltpu.sync_copy(acc2.at[pl.ds(row0, ROWS_PER_SUB)],
                    out_hbm.at[c, pl.ds(row0, ROWS_PER_SUB)])

  return sc_2layer


_BLK = 1000


def _mm1_body(x_ref, w_ref, o_ref):
  o_ref[...] = jnp.dot(x_ref[...], w_ref[...],
                       preferred_element_type=jnp.float32)


def _tc_mm1(x, w1):
  return pl.pallas_call(
      _mm1_body,
      out_shape=jax.ShapeDtypeStruct((N_NODES, D_HID), jnp.float32),
  )(x, w1)


def _sm_body(q_ref, w2_ref, b2_ref, o_ref):
  z = jnp.dot(q_ref[0, :N_NODES] + q_ref[1, :N_NODES], w2_ref[...],
              preferred_element_type=jnp.float32) + b2_ref[...]
  col = lax.broadcasted_iota(jnp.int32, z.shape, 1)
  mask = col < N_CLASSES
  zm = jnp.where(mask, z, -jnp.inf)
  m = jnp.max(zm, axis=1, keepdims=True)
  e = jnp.where(mask, jnp.exp(z - m), 0.0)
  ssum = jnp.sum(e, axis=1, keepdims=True)
  o_ref[...] = ((z - m) - jnp.log(ssum))[:, :N_CLASSES]


def _tc_final(qp, w2p, b2row):
  return pl.pallas_call(
      _sm_body,
      out_shape=jax.ShapeDtypeStruct((N_NODES, N_CLASSES), jnp.float32),
  )(qp, w2p, b2row)


def kernel(x, edge_index, edge_weight, W1, b1, W2, b2):
  E = edge_index.shape[1]
  K = -(-E // (NW * CHUNK))
  K = -(-K // NBUF) * NBUF
  e_pad = NW * K * CHUNK - E

  src = edge_index[0].astype(jnp.int32)
  dst = edge_index[1].astype(jnp.int32)
  ew = edge_weight.astype(jnp.float32)
  src3 = jnp.pad(src, (0, e_pad)).reshape(NW, K, CHUNK)
  dst3 = jnp.pad(dst, (0, e_pad)).reshape(NW, K, CHUNK)
  ewf = jnp.pad(ew, (0, e_pad)).reshape(NW, K * CHUNK)
  zeros = jnp.zeros((N_PAD, D_HID), jnp.float32)

  h1 = _tc_mm1(x, W1)
  qp, _ = _make_sc_2layer(K)(h1, src3, dst3, ewf, zeros, b1)
  w2p = jnp.zeros((D_HID, D_HID), jnp.float32).at[:, :N_CLASSES].set(W2)
  b2row = jnp.zeros((1, D_HID), jnp.float32).at[0, :N_CLASSES].set(b2)
  return _tc_final(qp, w2p, b2row)
